# baseline (speedup 1.0000x reference)
import functools

import jax
import jax.numpy as jnp
from jax import lax
from jax.experimental import pallas as pl
from jax.experimental.pallas import tpu as pltpu, tpu_sc as plsc

NC = 2
NS = 16
NW = NC * NS
CK = 128
NBUF = 4
NSC = 2


def _dense1_body(x_ref, w_ref, b_ref, o_ref):
    h = jnp.dot(x_ref[...], w_ref[...], preferred_element_type=jnp.float32)
    o_ref[...] = jnp.maximum(h + b_ref[...], 0.0)


def _dense1(x, W1, b1):
    n, d = x.shape
    h_ = W1.shape[1]
    blk = 2000
    return pl.pallas_call(
        _dense1_body,
        grid=(n // blk,),
        in_specs=[
            pl.BlockSpec((blk, d), lambda i: (i, 0)),
            pl.BlockSpec((d, h_), lambda i: (0, 0)),
            pl.BlockSpec((1, h_), lambda i: (0, 0)),
        ],
        out_specs=pl.BlockSpec((blk, h_), lambda i: (i, 0)),
        out_shape=jax.ShapeDtypeStruct((n, h_), jnp.float32),
    )(x, W1, b1.reshape(1, h_))


def _rsqrt16(v):
    i = plsc.bitcast(v, jnp.int32)
    y = plsc.bitcast(jnp.int32(0x5F3759DF) - (i >> 1), jnp.float32)
    for _ in range(3):
        y = y * (1.5 - 0.5 * v * y * y)
    return y


def _make_conv(n_pad, ep, h_):
    ch = ep // (NW * CK)
    rows_per = n_pad // NS
    zc = rows_per // CK
    ng = CK // 16
    mesh = plsc.VectorSubcoreMesh(core_axis_name="c", subcore_axis_name="s")

    @functools.partial(
        pl.kernel,
        mesh=mesh,
        out_type=(
            jax.ShapeDtypeStruct((NC, n_pad, h_), jnp.float32),
            jax.ShapeDtypeStruct((NC, n_pad), jnp.float32),
        ),
        scratch_types=[
            pltpu.VMEM((ch, CK), jnp.int32),
            pltpu.VMEM((ch, CK), jnp.int32),
            pltpu.VMEM((NBUF, CK, h_), jnp.float32),
            pltpu.VMEM((NBUF, CK, h_), jnp.float32),
            pltpu.VMEM((NSC, CK, h_), jnp.float32),
            pltpu.VMEM((NSC, CK), jnp.float32),
            pltpu.VMEM((16,), jnp.float32),
            pltpu.VMEM_SHARED((n_pad, h_), jnp.float32),
            pltpu.VMEM_SHARED((n_pad,), jnp.float32),
            pltpu.SemaphoreType.DMA((NBUF,)),
            pltpu.SemaphoreType.DMA((NBUF,)),
            pltpu.SemaphoreType.DMA((NSC,)),
            pltpu.SemaphoreType.DMA((NSC,)),
        ],
        compiler_params=pltpu.CompilerParams(use_tc_tiling_on_sc=False,
                                             needs_layout_passes=False),
    )
    def conv_k(h_hbm, src_hbm, dst_hbm, beta_hbm, z16_hbm, z1_hbm,
               acc_out, den_out,
               sidx, didx, hsb, hdb, cbuf, dbuf, bvecv,
               acc_sh, den_sh,
               gs, gd, sc, sd):
        cid = lax.axis_index("c")
        sid = lax.axis_index("s")
        wid = sid * NC + cid

        r0 = sid * rows_per
        pltpu.sync_copy(z16_hbm.at[pl.ds(r0, rows_per)], acc_sh.at[pl.ds(r0, rows_per)])
        pltpu.sync_copy(z1_hbm.at[pl.ds(r0, rows_per)], den_sh.at[pl.ds(r0, rows_per)])

        pltpu.sync_copy(src_hbm.at[pl.ds(wid * ch, ch)], sidx)
        pltpu.sync_copy(dst_hbm.at[pl.ds(wid * ch, ch)], didx)
        pltpu.sync_copy(beta_hbm, bvecv)
        bvec = bvecv[...]
        plsc.subcore_barrier()

        rows0 = lax.iota(jnp.int32, 16)

        def compute(hs, hd, cb, db):
            for g in range(ng):
                rows = rows0 + (16 * g)
                acol = []
                dot = jnp.zeros((16,), jnp.float32)
                ns = jnp.zeros((16,), jnp.float32)
                nd = jnp.zeros((16,), jnp.float32)
                for f in range(h_):
                    cols = jnp.full((16,), f, jnp.int32)
                    a = plsc.load_gather(hs, (rows, cols))
                    b = plsc.load_gather(hd, (rows, cols))
                    acol.append(a)
                    dot += a * b
                    ns += a * a
                    nd += b * b
                r = _rsqrt16(jnp.maximum(ns * nd, 1e-30))
                s = jnp.exp(bvec * dot * r)
                for f in range(h_):
                    cols = jnp.full((16,), f, jnp.int32)
                    plsc.store_scatter(cb, (rows, cols), s * acol[f])
                db[pl.ds(16 * g, 16)] = s

        def fire(j, b):
            pltpu.async_copy(h_hbm.at[sidx.at[j]], hsb.at[b], gs.at[b])
            pltpu.async_copy(h_hbm.at[didx.at[j]], hdb.at[b], gd.at[b])

        def wait_gather(j, b):
            pltpu.make_async_copy(h_hbm.at[sidx.at[j]], hsb.at[b], gs.at[b]).wait()
            pltpu.make_async_copy(h_hbm.at[didx.at[j]], hdb.at[b], gd.at[b]).wait()

        def scat(j, b):
            pltpu.async_copy(cbuf.at[b], acc_sh.at[didx.at[j]], sc.at[b], add=True)
            pltpu.async_copy(dbuf.at[b], den_sh.at[didx.at[j]], sd.at[b], add=True)

        def wait_scat(j, b):
            pltpu.make_async_copy(cbuf.at[b], acc_sh.at[didx.at[j]], sc.at[b]).wait()
            pltpu.make_async_copy(dbuf.at[b], den_sh.at[didx.at[j]], sd.at[b]).wait()

        for b in range(NBUF - 1):
            fire(b, b)

        def body(i, carry):
            for b in range(NBUF):
                j = NBUF * i + b
                wait_gather(j, b)
                @pl.when(j + NBUF - 1 < ch)
                def _():
                    fire(j + NBUF - 1, (b + NBUF - 1) % NBUF)
                sb = b % NSC
                @pl.when(j >= NSC)
                def _():
                    wait_scat(j, sb)
                compute(hsb.at[b], hdb.at[b], cbuf.at[sb], dbuf.at[sb])
                scat(j, sb)
            return carry

        lax.fori_loop(0, ch // NBUF, body, 0, unroll=False)
        for b in range(NSC):
            wait_scat(0, b)
        plsc.subcore_barrier()

        def wb(k, carry):
            r = sid * rows_per + k * CK
            pltpu.sync_copy(acc_sh.at[pl.ds(r, CK)], cbuf.at[0])
            pltpu.sync_copy(cbuf.at[0], acc_out.at[cid].at[pl.ds(r, CK)])
            pltpu.sync_copy(den_sh.at[pl.ds(r, CK)], dbuf.at[0])
            pltpu.sync_copy(dbuf.at[0], den_out.at[cid].at[pl.ds(r, CK)])
            return carry

        lax.fori_loop(0, zc, wb, 0, unroll=False)

    return conv_k


def _make_sc_combine(n_pad, h_):
    rows_w = n_pad // NW
    cchunk = 160
    nch = rows_w // cchunk
    mesh = plsc.VectorSubcoreMesh(core_axis_name="c", subcore_axis_name="s")

    @functools.partial(
        pl.kernel,
        mesh=mesh,
        out_type=jax.ShapeDtypeStruct((n_pad, h_), jnp.float32),
        scratch_types=[
            pltpu.VMEM((cchunk, h_), jnp.float32),
            pltpu.VMEM((cchunk, h_), jnp.float32),
            pltpu.VMEM((cchunk,), jnp.float32),
            pltpu.VMEM((cchunk,), jnp.float32),
            pltpu.VMEM((cchunk, h_), jnp.float32),
        ],
        compiler_params=pltpu.CompilerParams(use_tc_tiling_on_sc=False,
                                             needs_layout_passes=False),
    )
    def combine_k(acc_hbm, den_hbm, h_out, a0v, a1v, d0v, d1v, hv):
        cid = lax.axis_index("c")
        sid = lax.axis_index("s")
        wid = sid * NC + cid
        r0 = wid * rows_w
        rows0 = lax.iota(jnp.int32, 16)

        def body(k, carry):
            rb = r0 + k * cchunk
            pltpu.sync_copy(acc_hbm.at[0].at[pl.ds(rb, cchunk)], a0v)
            pltpu.sync_copy(acc_hbm.at[1].at[pl.ds(rb, cchunk)], a1v)
            pltpu.sync_copy(den_hbm.at[0].at[pl.ds(rb, cchunk)], d0v)
            pltpu.sync_copy(den_hbm.at[1].at[pl.ds(rb, cchunk)], d1v)
            for g in range(cchunk // 16):
                rows = rows0 + 16 * g
                dsum = d0v[pl.ds(16 * g, 16)] + d1v[pl.ds(16 * g, 16)]
                rinv = 1.0 / jnp.maximum(dsum, 1e-30)
                for f in range(h_):
                    cols = jnp.full((16,), f, jnp.int32)
                    col = (plsc.load_gather(a0v, (rows, cols))
                           + plsc.load_gather(a1v, (rows, cols))) * rinv
                    plsc.store_scatter(hv, (rows, cols), col)
            pltpu.sync_copy(hv, h_out.at[pl.ds(rb, cchunk)])
            return carry

        lax.fori_loop(0, nch, body, 0, unroll=False)

    return combine_k


def _combine_body(acc_ref, den_ref, o_ref):
    a = acc_ref[0] + acc_ref[1]
    d = jnp.maximum(den_ref[0] + den_ref[1], 1e-30)
    o_ref[...] = a / d


def _combine(acc, den):
    _, n_pad, h_ = acc.shape
    blk = 2048
    return pl.pallas_call(
        _combine_body,
        grid=(n_pad // blk,),
        in_specs=[
            pl.BlockSpec((2, blk, h_), lambda i: (0, i, 0)),
            pl.BlockSpec((2, blk, 1), lambda i: (0, i, 0)),
        ],
        out_specs=pl.BlockSpec((blk, h_), lambda i: (i, 0)),
        out_shape=jax.ShapeDtypeStruct((n_pad, h_), jnp.float32),
    )(acc, den)


def _final_body(h_ref, w_ref, b_ref, o_ref):
    logits = jnp.dot(h_ref[...], w_ref[...], preferred_element_type=jnp.float32) + b_ref[...]
    m = jnp.max(logits, axis=1, keepdims=True)
    lse = m + jnp.log(jnp.sum(jnp.exp(logits - m), axis=1, keepdims=True))
    o_ref[...] = logits - lse


def _final(h3, W4, b4, n):
    _, h_ = h3.shape
    c = W4.shape[1]
    blk = 2000
    return pl.pallas_call(
        _final_body,
        grid=(n // blk,),
        in_specs=[
            pl.BlockSpec((blk, h_), lambda i: (i, 0)),
            pl.BlockSpec((h_, c), lambda i: (0, 0)),
            pl.BlockSpec((1, c), lambda i: (0, 0)),
        ],
        out_specs=pl.BlockSpec((blk, c), lambda i: (i, 0)),
        out_shape=jax.ShapeDtypeStruct((n, c), jnp.float32),
    )(h3, W4, b4.reshape(1, c))


def kernel(x, edge_index, W1, b1, beta2, beta3, W4, b4):
    n, d = x.shape
    h_ = W1.shape[1]
    e = edge_index.shape[1]

    n_pad = 51200
    etot = e + n
    sup = NW * CK
    ep = ((etot + sup - 1) // sup) * sup
    pad = ep - etot

    loops = jnp.arange(n, dtype=jnp.int32)
    junk = jnp.full((pad,), n, dtype=jnp.int32)
    src = jnp.concatenate([edge_index[0], loops, junk]).reshape(ep // CK, CK)
    dst = jnp.concatenate([edge_index[1], loops, junk]).reshape(ep // CK, CK)

    h1 = _dense1(x, W1, b1)
    h1p = jnp.concatenate([h1, jnp.zeros((n_pad - n, h_), jnp.float32)], axis=0)

    z16 = jnp.zeros((n_pad, h_), jnp.float32)
    z1 = jnp.zeros((n_pad,), jnp.float32)

    conv_k = _make_conv(n_pad, ep, h_)
    combine_k = _make_sc_combine(n_pad, h_)

    def conv(hp, beta):
        return conv_k(hp, src, dst, jnp.full((16,), beta, jnp.float32),
                      z16, z1)

    acc1, den1 = conv(h1p, beta2)
    h2p = combine_k(acc1, den1)
    acc2, den2 = conv(h2p, beta3)
    h3 = combine_k(acc2, den2)
    return _final(h3, W4, b4, n)
